# baseline (speedup 1.0000x reference)
import jax
import jax.numpy as jnp
from jax import lax
from jax.experimental import pallas as pl
from jax.experimental.pallas import tpu as pltpu
from jax.experimental.pallas import tpu_sc as plsc

_V = 1000000
_D = 64
_B = 4096
_L = 200

_NC = 2
_NS = 16
_NW = _NC * _NS
_LANES = 16
_NBLK = _V // 128
_TAIL = _V - _NBLK * 128
_SLOT_PAIRS = (_NBLK // _NW + 2) // 2
_SEQ_W = _B // _NW

_MESH = dict(core_axis_name="c", subcore_axis_name="s",
             num_cores=_NC, num_subcores=_NS)
_PARAMS = pltpu.CompilerParams(
    use_tc_tiling_on_sc=True, needs_layout_passes=False)


def _wid():
  return lax.axis_index("s") * _NC + lax.axis_index("c")


def _row_bases():
  base = lax.iota(jnp.int32, _LANES)
  return [base + q * _LANES for q in range(8)]


_NB1 = 4


def _t_body(tblT, tailP, tblL, vin, vout, gsems, wsems):
  w = _wid()
  rows_q = _row_bases()
  zeros = jnp.zeros((_LANES,), jnp.int32)

  @pl.when(w == 0)
  def _tail():
    pltpu.sync_copy(tailP, tblL.at[pl.ds(_NBLK * 128, _TAIL)])

  def issue(k, p):
    b = w + k * _NW

    @pl.when(b < _NBLK)
    def _():
      pltpu.async_copy(tblT.at[:, pl.ds(b * 128, 128)], vin.at[p],
                       gsems.at[p])

  def process(k, p):
    b = w + k * _NW

    @pl.when(b < _NBLK)
    def _():
      pltpu.make_async_copy(
          tblT.at[:, pl.ds(0, 128)], vin.at[p], gsems.at[p]).wait()

      @pl.when(k >= _NB1)
      def _():
        pltpu.make_async_copy(
            vout.at[p], tblL.at[pl.ds(0, 128)], wsems.at[p]).wait()

      @plsc.parallel_loop(0, _D, unroll=8)
      def _d(d):
        dcol = zeros + d
        for q in range(8):
          v = vin[p, d, pl.ds(q * _LANES, _LANES)]
          plsc.store_scatter(vout.at[p], [rows_q[q], dcol], v)

      pltpu.async_copy(vout.at[p], tblL.at[pl.ds(b * 128, 128)], wsems.at[p])

  for p in range(_NB1 - 1):
    issue(p, p)

  @pl.loop(0, (_NBLK // _NW + _NB1) // _NB1 + 1)
  def _round(r):
    k0 = r * _NB1
    for half in range(_NB1):
      k = k0 + half
      p = half
      issue(k + _NB1 - 1, (half + _NB1 - 1) % _NB1)
      process(k, p)

  for p in range(_NB1):
    pltpu.make_async_copy(vout.at[p], tblL.at[pl.ds(0, 128)],
                          wsems.at[p]).wait()


_NB2 = 3


def _g_body(xT, tblL, posT, out, idx_v, posw, posv2, rows, slab, gsems, ssems):
  w = _wid()
  lane0 = w * _SEQ_W
  rows_q = _row_bases()
  zeros = jnp.zeros((_LANES,), jnp.int32)

  pltpu.sync_copy(xT.at[:, pl.ds(lane0, _SEQ_W)], idx_v)
  pltpu.sync_copy(posT, posw)

  @plsc.parallel_loop(0, _D, unroll=8)
  def _pd(d):
    dcol = zeros + d
    for q in range(13):
      o = min(q * _LANES, _L - _LANES)
      v = posw[d, pl.ds(o, _LANES)]
      lvec = rows_q[0] + o
      col = ((lvec & 1) << 6) + dcol
      plsc.store_scatter(posv2, [lvec >> 1, col], v)

  def issue(l, p):
    @pl.when(l < _L)
    def _():
      pltpu.async_copy(tblL.at[idx_v.at[l]], rows.at[p], gsems.at[p])

  def process(l, p):
    @pl.when(l < _L)
    def _():
      pltpu.make_async_copy(
          tblL.at[pl.ds(0, _SEQ_W)], rows.at[p], gsems.at[p]).wait()

      @pl.when(l >= _NB2)
      def _():
        pltpu.make_async_copy(
            slab.at[p], out.at[0, :, pl.ds(0, _SEQ_W)], ssems.at[p]).wait()

      lrow = l // 2
      lcol = (l % 2) * _D
      pvecs = [posv2[lrow, pl.ds(lcol + q * _LANES, _LANES)]
               for q in range(_D // _LANES)]

      @plsc.parallel_loop(0, _SEQ_W, unroll=8)
      def _tok(j):
        jcol = zeros + j
        for q in range(_D // _LANES):
          v = rows[p, j, pl.ds(q * _LANES, _LANES)]
          plsc.store_scatter(slab.at[p], [rows_q[q], jcol], v + pvecs[q])

      pltpu.async_copy(slab.at[p], out.at[l, :, pl.ds(lane0, _SEQ_W)],
                       ssems.at[p])

  for p in range(_NB2 - 1):
    issue(p, p)

  @pl.loop(0, _L // _NB2 + 1)
  def _round(r):
    l0 = r * _NB2
    for half in range(_NB2):
      l = l0 + half
      p = half
      issue(l + _NB2 - 1, (half + _NB2 - 1) % _NB2)
      process(l, p)

  for p in range(_NB2):
    pltpu.make_async_copy(
        slab.at[p], out.at[0, :, pl.ds(0, _SEQ_W)], ssems.at[p]).wait()


@jax.jit
def _run(x, token_table, pos_table):
  mesh = plsc.VectorSubcoreMesh(**_MESH)
  tblT = token_table.T
  xT = x.T
  posT = pos_table.T
  tailP = jnp.pad(token_table[_NBLK * 128:], ((0, 0), (0, 128 - _D)))

  t_kern = pl.kernel(
      _t_body,
      out_type=jax.ShapeDtypeStruct((_V, 128), jnp.float32),
      mesh=mesh,
      scratch_types=[
          pltpu.VMEM((_NB1, _D, 128), jnp.float32),
          pltpu.VMEM((_NB1, 128, 128), jnp.float32),
          pltpu.SemaphoreType.DMA((_NB1,)),
          pltpu.SemaphoreType.DMA((_NB1,)),
      ],
      compiler_params=_PARAMS,
  )
  tblL = t_kern(tblT, tailP)

  g_kern = pl.kernel(
      _g_body,
      out_type=jax.ShapeDtypeStruct((_L, _D, _B), jnp.float32),
      mesh=mesh,
      scratch_types=[
          pltpu.VMEM((_L, _SEQ_W), jnp.int32),
          pltpu.VMEM((_D, _L), jnp.float32),
          pltpu.VMEM((_L // 2, 128), jnp.float32),
          pltpu.VMEM((_NB2, _SEQ_W, 128), jnp.float32),
          pltpu.VMEM((_NB2, _D, _SEQ_W), jnp.float32),
          pltpu.SemaphoreType.DMA((_NB2,)),
          pltpu.SemaphoreType.DMA((_NB2,)),
      ],
      compiler_params=_PARAMS,
  )
  outK = g_kern(xT, tblL, posT)
  return jnp.transpose(outK, (2, 0, 1))


def kernel(x, token_table, pos_table):
  return _run(x, token_table, pos_table[:_L])


# speedup vs baseline: 1.2676x; 1.2676x over previous
import jax
import jax.numpy as jnp
from jax import lax
from jax.experimental import pallas as pl
from jax.experimental.pallas import tpu as pltpu
from jax.experimental.pallas import tpu_sc as plsc

_VOCAB = 1000000
_D = 64
_B = 4096
_L = 200

_NC = 2
_NS = 16
_NW = _NC * _NS
_SEQ_W = _B // _NW
_SPC = 2
_NCH = _SEQ_W // _SPC
_LANES = 16
_SPLITS = ((0, 128), (128, 72))


def _sc_body(x, tbl, posf, out, idx_all, rows_v, pos_v,
             gsem0, gsem1, ssem0, ssem1):
  cid = lax.axis_index("c")
  sid = lax.axis_index("s")
  wid = sid * _NC + cid
  seq_base = wid * _SEQ_W

  gsems = (gsem0, gsem1)
  ssems = (ssem0, ssem1)

  pltpu.sync_copy(posf, pos_v)
  pltpu.sync_copy(x.at[pl.ds(seq_base, _SEQ_W)], idx_all)

  def issue_gather(c, b):
    for j in range(_SPC):
      for (o, n) in _SPLITS:
        pltpu.async_copy(
            tbl.at[idx_all.at[c * _SPC + j, pl.ds(o, n)]],
            rows_v.at[b, j, pl.ds(o, n)], gsems[b])

  def wait_gather(b):
    pltpu.make_async_copy(
        out.at[pl.ds(0, _SPC)], rows_v.at[b], gsems[b]).wait()

  def issue_store(c, b):
    pltpu.async_copy(rows_v.at[b],
                     out.at[pl.ds(seq_base + c * _SPC, _SPC)], ssems[b])

  def wait_store(b):
    pltpu.make_async_copy(rows_v.at[b], out.at[pl.ds(0, _SPC)],
                          ssems[b]).wait()

  def add_pos(b):
    @pl.loop(0, _L, unroll=2)
    def _row(r):
      for k in range(_D // _LANES):
        v = pos_v[r, pl.ds(k * _LANES, _LANES)]
        for j in range(_SPC):
          plsc.addupdate(rows_v.at[b, j, r, pl.ds(k * _LANES, _LANES)], v)

  issue_gather(0, 0)

  @pl.loop(0, _NCH // 2)
  def _pair(c2):
    c0 = c2 * 2
    for half in range(2):
      cc = c0 + half
      b = half
      nb = 1 - half

      @pl.when(cc > 0)
      def _():
        wait_store(nb)

      @pl.when(cc + 1 < _NCH)
      def _():
        issue_gather(cc + 1, nb)

      wait_gather(b)
      add_pos(b)
      issue_store(cc, b)

  wait_store(1)


@jax.jit
def _run(x, token_table, pos_table):
  mesh = plsc.VectorSubcoreMesh(
      core_axis_name="c", subcore_axis_name="s",
      num_cores=_NC, num_subcores=_NS)
  kern = pl.kernel(
      _sc_body,
      out_type=jax.ShapeDtypeStruct((_B, _L, _D), jnp.float32),
      mesh=mesh,
      scratch_types=[
          pltpu.VMEM((_SEQ_W, _L), jnp.int32),
          pltpu.VMEM((2, _SPC, _L, _D), jnp.float32),
          pltpu.VMEM((_L, _D), jnp.float32),
          pltpu.SemaphoreType.DMA,
          pltpu.SemaphoreType.DMA,
          pltpu.SemaphoreType.DMA,
          pltpu.SemaphoreType.DMA,
      ],
      compiler_params=pltpu.CompilerParams(use_tc_tiling_on_sc=False),
  )
  return kern(x, token_table, pos_table)


def kernel(x, token_table, pos_table):
  return _run(x, token_table, pos_table[:_L])
